# Initial kernel scaffold; baseline (speedup 1.0000x reference)
import jax
import jax.numpy as jnp
from jax.experimental import pallas as pl


def kernel(inputs, W_attn, b_attn, W_gate, b_gate, W1, b1, W2, b2):
    raise NotImplementedError("write your pallas kernel here")


# speedup vs baseline: 13.8195x; 13.8195x over previous
import functools

import jax
import jax.numpy as jnp
from jax import lax
from jax.experimental import pallas as pl
from jax.experimental.pallas import tpu as pltpu

B, S, D = 2, 2048, 768
E, U1, U2, TOP_K = 8, 768, 768, 2


def _gate_kernel(x_ref, wa_ref, wg_ref, bg_ref, idx_ref, w_ref):
    x = x_ref[0]
    wa = wa_ref[...]
    logits = jnp.sum(x * wa, axis=1, keepdims=True)
    m = jnp.max(logits)
    e = jnp.exp(logits - m)
    scores = e / jnp.sum(e)
    attn = jnp.sum(x * scores, axis=0, keepdims=True)
    glog = jnp.dot(attn, wg_ref[...],
                   preferred_element_type=jnp.float32) + bg_ref[...]
    gm = jnp.max(glog)
    ge = jnp.exp(glog - gm)
    g = ge / jnp.sum(ge)
    iota = lax.broadcasted_iota(jnp.int32, (1, E), 1)
    v1 = jnp.max(g)
    i1 = jnp.min(jnp.where(g == v1, iota, E))
    g2 = jnp.where(iota == i1, -1.0, g)
    v2 = jnp.max(g2)
    i2 = jnp.min(jnp.where(g2 == v2, iota, E))
    s = v1 + v2 + 1e-9
    b = pl.program_id(0)
    idx_ref[2 * b] = i1
    idx_ref[2 * b + 1] = i2
    w_ref[2 * b] = v1 / s
    w_ref[2 * b + 1] = v2 / s


def _gelu_exact(x):
    return x * 0.5 * (1.0 + lax.erf(x * 0.7071067811865476))


def _ffn_kernel(idx_ref, w_ref, x_ref, w1_ref, b1_ref, w2_ref, b2_ref,
                out_ref):
    del idx_ref
    b = pl.program_id(0)
    k = pl.program_id(1)
    x = x_ref[0]
    h = jnp.dot(x, w1_ref[0], preferred_element_type=jnp.float32)
    h = _gelu_exact(h + b1_ref[0])
    o = jnp.dot(h, w2_ref[0], preferred_element_type=jnp.float32)
    o = _gelu_exact(o + b2_ref[0])
    w = w_ref[2 * b + k]

    @pl.when(k == 0)
    def _():
        out_ref[0] = w * o

    @pl.when(k == 1)
    def _():
        out_ref[0] += w * o


@jax.jit
def kernel(inputs, W_attn, b_attn, W_gate, b_gate, W1, b1, W2, b2):
    del b_attn
    wa_t = W_attn.reshape(1, D)
    bg = b_gate.reshape(1, E)

    idx, w = pl.pallas_call(
        _gate_kernel,
        grid=(B,),
        in_specs=[
            pl.BlockSpec((1, S, D), lambda b: (b, 0, 0)),
            pl.BlockSpec((1, D), lambda b: (0, 0)),
            pl.BlockSpec((D, E), lambda b: (0, 0)),
            pl.BlockSpec((1, E), lambda b: (0, 0)),
        ],
        out_specs=[
            pl.BlockSpec(memory_space=pltpu.SMEM),
            pl.BlockSpec(memory_space=pltpu.SMEM),
        ],
        out_shape=[
            jax.ShapeDtypeStruct((B * TOP_K,), jnp.int32),
            jax.ShapeDtypeStruct((B * TOP_K,), jnp.float32),
        ],
    )(inputs, wa_t, W_gate, bg)

    b1r = b1.reshape(E, 1, U1)
    b2r = b2.reshape(E, 1, U2)

    grid_spec = pltpu.PrefetchScalarGridSpec(
        num_scalar_prefetch=2,
        grid=(B, TOP_K),
        in_specs=[
            pl.BlockSpec((1, S, D), lambda b, k, idx, w: (b, 0, 0)),
            pl.BlockSpec((1, D, U1),
                         lambda b, k, idx, w: (idx[2 * b + k], 0, 0)),
            pl.BlockSpec((1, 1, U1),
                         lambda b, k, idx, w: (idx[2 * b + k], 0, 0)),
            pl.BlockSpec((1, U1, U2),
                         lambda b, k, idx, w: (idx[2 * b + k], 0, 0)),
            pl.BlockSpec((1, 1, U2),
                         lambda b, k, idx, w: (idx[2 * b + k], 0, 0)),
        ],
        out_specs=pl.BlockSpec((1, S, U2), lambda b, k, idx, w: (b, 0, 0)),
    )

    out = pl.pallas_call(
        _ffn_kernel,
        grid_spec=grid_spec,
        out_shape=jax.ShapeDtypeStruct((B, S, U2), jnp.float32),
    )(idx, w, inputs, W1, b1r, W2, b2r)
    return out
